# baseline (speedup 1.0000x reference)
import jax
import jax.numpy as jnp
from jax.experimental import pallas as pl
from jax.experimental.pallas import tpu as pltpu

_N = 131072
_H = 32
_K = 512
_NB = 2048
_RC = 8


def _body(cm2_ref, x_ref, c2_ref, o_ref):
    cm2 = cm2_ref[...]
    xt = x_ref[...].T
    dt = jax.lax.dot_general(
        cm2, xt, (((1,), (0,)), ((), ())),
        preferred_element_type=jnp.float32) + c2_ref[...]

    best = dt[0:_RC, :]
    bi = jnp.zeros((_RC, _NB), jnp.float32)
    for c in range(1, _K // _RC):
        blk = dt[c * _RC:(c + 1) * _RC, :]
        take = blk < best
        best = jnp.where(take, blk, best)
        bi = jnp.where(take, jnp.float32(c), bi)
    srow = jax.lax.broadcasted_iota(jnp.int32, (_RC, _NB), 0).astype(jnp.float32)
    rowval = bi * jnp.float32(_RC) + srow
    m = jnp.min(best, axis=0, keepdims=True)
    idx = jnp.min(jnp.where(best <= m, rowval, jnp.float32(2 * _K)),
                  axis=0, keepdims=True)
    o_ref[...] = idx.astype(jnp.int32)


def kernel(x, cluster_centers):
    cm2 = -2.0 * cluster_centers
    c2 = jnp.sum(cluster_centers * cluster_centers, axis=1)[:, None]
    grid = (_N // _NB,)
    out = pl.pallas_call(
        _body,
        grid=grid,
        in_specs=[
            pl.BlockSpec((_K, _H), lambda i: (0, 0)),
            pl.BlockSpec((_NB, _H), lambda i: (i, 0)),
            pl.BlockSpec((_K, 1), lambda i: (0, 0)),
        ],
        out_specs=pl.BlockSpec((1, _NB), lambda i: (0, i)),
        out_shape=jax.ShapeDtypeStruct((1, _N), jnp.int32),
        compiler_params=pltpu.CompilerParams(
            dimension_semantics=("arbitrary",)),
    )(cm2, x, c2)
    return out.reshape(_N, 1)


# speedup vs baseline: 2.2487x; 2.2487x over previous
import jax
import jax.numpy as jnp
from jax.experimental import pallas as pl
from jax.experimental.pallas import tpu as pltpu

_N = 131072
_H = 32
_K = 512
_NB = 4096
_RC = 8


def _body(cm2_ref, xt_ref, c2_ref, o_ref):
    cm2 = cm2_ref[...]
    dt = jax.lax.dot_general(
        cm2, xt_ref[...], (((1,), (0,)), ((), ())),
        preferred_element_type=jnp.float32) + c2_ref[...]

    best = dt[0:_RC, :]
    bi = jnp.zeros((_RC, _NB), jnp.float32)
    for c in range(1, _K // _RC):
        blk = dt[c * _RC:(c + 1) * _RC, :]
        take = blk < best
        best = jnp.where(take, blk, best)
        bi = jnp.where(take, jnp.float32(c), bi)
    srow = jax.lax.broadcasted_iota(jnp.int32, (_RC, _NB), 0).astype(jnp.float32)
    rowval = bi * jnp.float32(_RC) + srow
    m = jnp.min(best, axis=0, keepdims=True)
    idx = jnp.min(jnp.where(best <= m, rowval, jnp.float32(2 * _K)),
                  axis=0, keepdims=True)
    o_ref[...] = idx.astype(jnp.int32)


def kernel(x, cluster_centers):
    cm2 = -2.0 * cluster_centers
    c2 = jnp.sum(cluster_centers * cluster_centers, axis=1)[:, None]
    xt = x.T
    grid = (_N // _NB,)
    out = pl.pallas_call(
        _body,
        grid=grid,
        in_specs=[
            pl.BlockSpec((_K, _H), lambda i: (0, 0)),
            pl.BlockSpec((_H, _NB), lambda i: (0, i)),
            pl.BlockSpec((_K, 1), lambda i: (0, 0)),
        ],
        out_specs=pl.BlockSpec((1, _NB), lambda i: (0, i)),
        out_shape=jax.ShapeDtypeStruct((1, _N), jnp.int32),
        compiler_params=pltpu.CompilerParams(
            dimension_semantics=("arbitrary",)),
    )(cm2, xt, c2)
    return out.reshape(_N, 1)


# speedup vs baseline: 2.3582x; 1.0487x over previous
import jax
import jax.numpy as jnp
from jax.experimental import pallas as pl
from jax.experimental.pallas import tpu as pltpu

_N = 131072
_H = 32
_K = 512
_NB = 8192
_RC = 8


def _body(cm2_ref, xt_ref, c2_ref, o_ref):
    cm2 = cm2_ref[...]
    dt = jax.lax.dot_general(
        cm2, xt_ref[...], (((1,), (0,)), ((), ())),
        preferred_element_type=jnp.float32) + c2_ref[...]

    best = dt[0:_RC, :]
    bi = jnp.zeros((_RC, _NB), jnp.float32)
    for c in range(1, _K // _RC):
        blk = dt[c * _RC:(c + 1) * _RC, :]
        take = blk < best
        best = jnp.where(take, blk, best)
        bi = jnp.where(take, jnp.float32(c), bi)
    srow = jax.lax.broadcasted_iota(jnp.int32, (_RC, _NB), 0).astype(jnp.float32)
    rowval = bi * jnp.float32(_RC) + srow
    m = jnp.min(best, axis=0, keepdims=True)
    idx = jnp.min(jnp.where(best <= m, rowval, jnp.float32(2 * _K)),
                  axis=0, keepdims=True)
    o_ref[...] = idx.astype(jnp.int32)


def kernel(x, cluster_centers):
    cm2 = -2.0 * cluster_centers
    c2 = jnp.sum(cluster_centers * cluster_centers, axis=1)[:, None]
    xt = x.T
    grid = (_N // _NB,)
    out = pl.pallas_call(
        _body,
        grid=grid,
        in_specs=[
            pl.BlockSpec((_K, _H), lambda i: (0, 0)),
            pl.BlockSpec((_H, _NB), lambda i: (0, i)),
            pl.BlockSpec((_K, 1), lambda i: (0, 0)),
        ],
        out_specs=pl.BlockSpec((1, _NB), lambda i: (0, i)),
        out_shape=jax.ShapeDtypeStruct((1, _N), jnp.int32),
        compiler_params=pltpu.CompilerParams(
            dimension_semantics=("arbitrary",)),
    )(cm2, xt, c2)
    return out.reshape(_N, 1)


# speedup vs baseline: 2.3943x; 1.0153x over previous
import jax
import jax.numpy as jnp
from jax.experimental import pallas as pl
from jax.experimental.pallas import tpu as pltpu

_N = 131072
_H = 32
_K = 512
_NB = 16384
_RC = 8


def _body(cm2_ref, xt_ref, c2_ref, o_ref):
    cm2 = cm2_ref[...]
    dt = jax.lax.dot_general(
        cm2, xt_ref[...], (((1,), (0,)), ((), ())),
        preferred_element_type=jnp.float32) + c2_ref[...]

    best = dt[0:_RC, :]
    bi = jnp.zeros((_RC, _NB), jnp.float32)
    for c in range(1, _K // _RC):
        blk = dt[c * _RC:(c + 1) * _RC, :]
        take = blk < best
        best = jnp.where(take, blk, best)
        bi = jnp.where(take, jnp.float32(c), bi)
    srow = jax.lax.broadcasted_iota(jnp.int32, (_RC, _NB), 0).astype(jnp.float32)
    rowval = bi * jnp.float32(_RC) + srow
    m = jnp.min(best, axis=0, keepdims=True)
    idx = jnp.min(jnp.where(best <= m, rowval, jnp.float32(2 * _K)),
                  axis=0, keepdims=True)
    o_ref[...] = idx.astype(jnp.int32)


def kernel(x, cluster_centers):
    cm2 = -2.0 * cluster_centers
    c2 = jnp.sum(cluster_centers * cluster_centers, axis=1)[:, None]
    xt = x.T
    grid = (_N // _NB,)
    out = pl.pallas_call(
        _body,
        grid=grid,
        in_specs=[
            pl.BlockSpec((_K, _H), lambda i: (0, 0)),
            pl.BlockSpec((_H, _NB), lambda i: (0, i)),
            pl.BlockSpec((_K, 1), lambda i: (0, 0)),
        ],
        out_specs=pl.BlockSpec((1, _NB), lambda i: (0, i)),
        out_shape=jax.ShapeDtypeStruct((1, _N), jnp.int32),
        compiler_params=pltpu.CompilerParams(
            dimension_semantics=("arbitrary",)),
    )(cm2, xt, c2)
    return out.reshape(_N, 1)


# speedup vs baseline: 2.3951x; 1.0003x over previous
import jax
import jax.numpy as jnp
from jax.experimental import pallas as pl
from jax.experimental.pallas import tpu as pltpu

_N = 131072
_H = 32
_K = 512
_NB = 32768
_RC = 8


def _body(cm2_ref, xt_ref, c2_ref, o_ref):
    cm2 = cm2_ref[...]
    dt = jax.lax.dot_general(
        cm2, xt_ref[...], (((1,), (0,)), ((), ())),
        preferred_element_type=jnp.float32) + c2_ref[...]

    best = dt[0:_RC, :]
    bi = jnp.zeros((_RC, _NB), jnp.float32)
    for c in range(1, _K // _RC):
        blk = dt[c * _RC:(c + 1) * _RC, :]
        take = blk < best
        best = jnp.where(take, blk, best)
        bi = jnp.where(take, jnp.float32(c), bi)
    srow = jax.lax.broadcasted_iota(jnp.int32, (_RC, _NB), 0).astype(jnp.float32)
    rowval = bi * jnp.float32(_RC) + srow
    m = jnp.min(best, axis=0, keepdims=True)
    idx = jnp.min(jnp.where(best <= m, rowval, jnp.float32(2 * _K)),
                  axis=0, keepdims=True)
    o_ref[...] = idx.astype(jnp.int32)


def kernel(x, cluster_centers):
    cm2 = -2.0 * cluster_centers
    c2 = jnp.sum(cluster_centers * cluster_centers, axis=1)[:, None]
    xt = x.T
    grid = (_N // _NB,)
    out = pl.pallas_call(
        _body,
        grid=grid,
        in_specs=[
            pl.BlockSpec((_K, _H), lambda i: (0, 0)),
            pl.BlockSpec((_H, _NB), lambda i: (0, i)),
            pl.BlockSpec((_K, 1), lambda i: (0, 0)),
        ],
        out_specs=pl.BlockSpec((1, _NB), lambda i: (0, i)),
        out_shape=jax.ShapeDtypeStruct((1, _N), jnp.int32),
        compiler_params=pltpu.CompilerParams(
            dimension_semantics=("arbitrary",)),
    )(cm2, xt, c2)
    return out.reshape(_N, 1)
